# baseline (speedup 1.0000x reference)
import jax
import jax.numpy as jnp
from jax import lax
from jax.experimental import pallas as pl
from jax.experimental.pallas import tpu as pltpu
from jax.experimental.pallas import tpu_sc as plsc

D = 512
EPS = 1e-5
NW = 32
CHUNK = 64


def _make_sc_gather(n_tok):
    tok_per_w = n_tok // NW
    n_chunks = tok_per_w // CHUNK
    mesh = plsc.VectorSubcoreMesh(core_axis_name="c", subcore_axis_name="s")

    NBUF = 3

    def body(x_hbm, table_hbm, out_hbm, idx_v, b0, b1, b2, g0, g1, g2,
             o0, o1, o2):
        wid = lax.axis_index("s") * 2 + lax.axis_index("c")
        base = wid * tok_per_w
        pltpu.sync_copy(x_hbm.at[wid], idx_v)

        bufs = (b0, b1, b2)
        gsems = (g0, g1, g2)
        osems = (o0, o1, o2)

        def gather(c):
            return pltpu.async_copy(
                table_hbm.at[idx_v.at[c]], bufs[c % NBUF], gsems[c % NBUF])

        def put(c):
            return pltpu.async_copy(
                bufs[c % NBUF], out_hbm.at[pl.ds(base + c * CHUNK, CHUNK)],
                osems[c % NBUF])

        gathers = [None] * n_chunks
        puts = [None] * n_chunks
        for i in range(NBUF - 1):
            gathers[i] = gather(i)
        for c in range(n_chunks):
            nxt = c + NBUF - 1
            if nxt < n_chunks:
                if nxt - NBUF >= 0:
                    puts[nxt - NBUF].wait()
                gathers[nxt] = gather(nxt)
            gathers[c].wait()
            puts[c] = put(c)
        for c in range(n_chunks - NBUF, n_chunks):
            puts[c].wait()

    return pl.kernel(
        body,
        out_type=jax.ShapeDtypeStruct((n_tok, D), jnp.float32),
        mesh=mesh,
        scratch_types=[
            pltpu.VMEM((n_chunks, CHUNK), jnp.int32),
            pltpu.VMEM((CHUNK, D), jnp.float32),
            pltpu.VMEM((CHUNK, D), jnp.float32),
            pltpu.VMEM((CHUNK, D), jnp.float32),
            pltpu.SemaphoreType.DMA,
            pltpu.SemaphoreType.DMA,
            pltpu.SemaphoreType.DMA,
            pltpu.SemaphoreType.DMA,
            pltpu.SemaphoreType.DMA,
            pltpu.SemaphoreType.DMA,
        ],
    )


def _tc_ln_body(rows_ref, pos_ref, mod_ref, g_ref, b_ref, o_ref):
    h = rows_ref[...] + pos_ref[...][None] + mod_ref[...][None]
    mean = jnp.mean(h, axis=-1, keepdims=True)
    meansq = jnp.mean(h * h, axis=-1, keepdims=True)
    var = meansq - mean * mean
    scale = lax.rsqrt(var + EPS) * g_ref[...][None]
    shift = b_ref[...][None] - mean * scale
    o_ref[...] = h * scale + shift


def _tc_ln(rows3d, pos2d, mod_row, g2d, b2d, ts, bsz, seq):
    return pl.pallas_call(
        _tc_ln_body,
        grid=(seq // ts,),
        in_specs=[
            pl.BlockSpec((bsz, ts, D), lambda j: (0, j, 0)),
            pl.BlockSpec((ts, D), lambda j: (j, 0)),
            pl.BlockSpec((1, D), lambda j: (0, 0)),
            pl.BlockSpec((1, D), lambda j: (0, 0)),
            pl.BlockSpec((1, D), lambda j: (0, 0)),
        ],
        out_specs=pl.BlockSpec((bsz, ts, D), lambda j: (0, j, 0)),
        out_shape=jax.ShapeDtypeStruct((bsz, seq, D), jnp.float32),
        input_output_aliases={0: 0},
    )(rows3d, pos2d, mod_row, g2d, b2d)


def kernel(x, token_table, pos_emb, mod_table, gamma, beta):
    bsz, seq = x.shape
    n_tok = bsz * seq
    n_chunks = n_tok // NW // CHUNK
    x_arr = x.astype(jnp.int32).reshape(NW, n_chunks, CHUNK)
    rows = _make_sc_gather(n_tok)(x_arr, token_table)
    pos2d = pos_emb.reshape(seq, D)
    return _tc_ln(rows.reshape(bsz, seq, D), pos2d, mod_table[0:1],
                  gamma.reshape(1, D), beta.reshape(1, D), 1024, bsz, seq)


# speedup vs baseline: 1.0192x; 1.0192x over previous
import jax
import jax.numpy as jnp
from jax import lax
from jax.experimental import pallas as pl
from jax.experimental.pallas import tpu as pltpu
from jax.experimental.pallas import tpu_sc as plsc

D = 512
EPS = 1e-5
NW = 32
CHUNK = 64


def _make_sc_gather(n_tok):
    tok_per_w = n_tok // NW
    n_chunks = tok_per_w // CHUNK
    mesh = plsc.VectorSubcoreMesh(core_axis_name="c", subcore_axis_name="s")

    NBUF = 3

    def body(x_hbm, table_hbm, out_hbm, idx_v, b0, b1, b2, g0, g1, g2,
             o0, o1, o2):
        wid = lax.axis_index("s") * 2 + lax.axis_index("c")
        base = wid * tok_per_w
        pltpu.sync_copy(x_hbm.at[wid], idx_v)

        bufs = (b0, b1, b2)
        gsems = (g0, g1, g2)
        osems = (o0, o1, o2)

        def gather(c):
            return pltpu.async_copy(
                table_hbm.at[idx_v.at[c]], bufs[c % NBUF], gsems[c % NBUF])

        def put(c):
            return pltpu.async_copy(
                bufs[c % NBUF], out_hbm.at[pl.ds(base + c * CHUNK, CHUNK)],
                osems[c % NBUF])

        gathers = [None] * n_chunks
        puts = [None] * n_chunks
        for i in range(NBUF - 1):
            gathers[i] = gather(i)
        for c in range(n_chunks):
            nxt = c + NBUF - 1
            if nxt < n_chunks:
                if nxt - NBUF >= 0:
                    puts[nxt - NBUF].wait()
                gathers[nxt] = gather(nxt)
            gathers[c].wait()
            puts[c] = put(c)
        for c in range(n_chunks - NBUF, n_chunks):
            puts[c].wait()

    return pl.kernel(
        body,
        out_type=jax.ShapeDtypeStruct((n_tok, D), jnp.float32),
        mesh=mesh,
        scratch_types=[
            pltpu.VMEM((n_chunks, CHUNK), jnp.int32),
            pltpu.VMEM((CHUNK, D), jnp.float32),
            pltpu.VMEM((CHUNK, D), jnp.float32),
            pltpu.VMEM((CHUNK, D), jnp.float32),
            pltpu.SemaphoreType.DMA,
            pltpu.SemaphoreType.DMA,
            pltpu.SemaphoreType.DMA,
            pltpu.SemaphoreType.DMA,
            pltpu.SemaphoreType.DMA,
            pltpu.SemaphoreType.DMA,
        ],
    )


def _tc_ln_body(rows_ref, pos_ref, mod_ref, g_ref, b_ref, o_ref):
    h = rows_ref[...] + pos_ref[...][None] + mod_ref[...][None]
    mean = jnp.mean(h, axis=-1, keepdims=True)
    meansq = jnp.mean(h * h, axis=-1, keepdims=True)
    var = meansq - mean * mean
    scale = lax.rsqrt(var + EPS) * g_ref[...][None]
    shift = b_ref[...][None] - mean * scale
    o_ref[...] = h * scale + shift


def _tc_ln(rows3d, pos2d, mod_row, g2d, b2d, ts, bsz, seq):
    return pl.pallas_call(
        _tc_ln_body,
        grid=(seq // ts,),
        in_specs=[
            pl.BlockSpec((bsz, ts, D), lambda j: (0, j, 0)),
            pl.BlockSpec((ts, D), lambda j: (j, 0)),
            pl.BlockSpec((1, D), lambda j: (0, 0)),
            pl.BlockSpec((1, D), lambda j: (0, 0)),
            pl.BlockSpec((1, D), lambda j: (0, 0)),
        ],
        out_specs=pl.BlockSpec((bsz, ts, D), lambda j: (0, j, 0)),
        out_shape=jax.ShapeDtypeStruct((bsz, seq, D), jnp.float32),
    )(rows3d, pos2d, mod_row, g2d, b2d)


def kernel(x, token_table, pos_emb, mod_table, gamma, beta):
    bsz, seq = x.shape
    n_tok = bsz * seq
    n_chunks = n_tok // NW // CHUNK
    x_arr = x.astype(jnp.int32).reshape(NW, n_chunks, CHUNK)
    rows = _make_sc_gather(n_tok)(x_arr, token_table)
    pos2d = pos_emb.reshape(seq, D)
    return _tc_ln(rows.reshape(bsz, seq, D), pos2d, mod_table[0:1],
                  gamma.reshape(1, D), beta.reshape(1, D), 1024, bsz, seq)
